# baseline (speedup 1.0000x reference)
import functools

import jax
import jax.numpy as jnp
from jax import lax
from jax.experimental import pallas as pl
from jax.experimental.pallas import tpu as pltpu
from jax.experimental.pallas import tpu_sc as plsc

VOCAB = 1000000
EMBED = 64
BATCH = 4096
SEQ = 200
NTOK = BATCH * SEQ

_NC = 2
_NS = 16
_NW = _NC * _NS
_RPW = BATCH // _NW
_BPW = _RPW * SEQ
_GRP = 2
_NG = _RPW // _GRP


def _make_emb():
    mesh = plsc.VectorSubcoreMesh(core_axis_name="c", subcore_axis_name="s")

    @functools.partial(
        pl.kernel,
        mesh=mesh,
        out_type=jax.ShapeDtypeStruct((BATCH, SEQ, EMBED), jnp.float32),
        compiler_params=pltpu.CompilerParams(use_tc_tiling_on_sc=False),
        scratch_types=[
            pltpu.VMEM((_BPW,), jnp.int32),
            pltpu.VMEM((2 * _GRP, SEQ, EMBED), jnp.float32),
            pltpu.SemaphoreType.DMA,
            pltpu.SemaphoreType.DMA,
        ],
    )
    def emb(idx_hbm, table_hbm, out_hbm, idx_v, bufs, gsem, wsem):
        wid = lax.axis_index("s") * _NC + lax.axis_index("c")
        base = wid * _BPW
        row0 = wid * _RPW
        pltpu.sync_copy(idx_hbm.at[pl.ds(base, _BPW)], idx_v)

        def gstart(r, b):
            pltpu.make_async_copy(
                table_hbm.at[idx_v.at[pl.ds(r * SEQ, SEQ)]], bufs.at[b],
                gsem).start()

        def gwait(b):
            pltpu.make_async_copy(
                table_hbm.at[idx_v.at[pl.ds(0, SEQ)]], bufs.at[b],
                gsem).wait()

        def wstart(r, b):
            pltpu.make_async_copy(
                bufs.at[b], out_hbm.at[row0 + r], wsem).start()

        def wwait(b):
            pltpu.make_async_copy(
                bufs.at[b], out_hbm.at[row0], wsem).wait()

        def g_start(g, bb):
            for b in range(_GRP):
                gstart(g * _GRP + b, bb + b)

        def g_wait(bb):
            for b in range(_GRP):
                gwait(bb + b)

        def w_start(g, bb):
            for b in range(_GRP):
                wstart(g * _GRP + b, bb + b)

        def w_wait(bb):
            for b in range(_GRP):
                wwait(bb + b)

        g_start(0, 0)
        g_wait(0)
        w_start(0, 0)
        g_start(1, _GRP)

        def body(j, carry):
            g1 = 2 * j + 1
            g_wait(_GRP)
            w_wait(0)
            w_start(g1, _GRP)
            g_start(g1 + 1, 0)
            g2 = g1 + 1
            g_wait(0)
            w_wait(_GRP)
            w_start(g2, 0)
            g_start(g2 + 1, _GRP)
            return carry

        lax.fori_loop(0, (_NG - 2) // 2, body, 0)
        g_wait(_GRP)
        w_wait(0)
        w_start(_NG - 1, _GRP)
        w_wait(_GRP)

    return emb


_emb = _make_emb()


def kernel(input, word_embed):
    idx = input.reshape(-1).astype(jnp.int32)
    return _emb(idx, word_embed)


# speedup vs baseline: 1.2613x; 1.2613x over previous
import functools

import jax
import jax.numpy as jnp
from jax import lax
from jax.experimental import pallas as pl
from jax.experimental.pallas import tpu as pltpu
from jax.experimental.pallas import tpu_sc as plsc
from jax.experimental.layout import Layout, with_layout_constraint

VOCAB = 1000000
EMBED = 64
BATCH = 4096
SEQ = 200
NTOK = BATCH * SEQ

_NC = 2
_NS = 16
_NW = _NC * _NS
_RPW = BATCH // _NW
_BPW = _RPW * SEQ
_GRP = 2
_NG = _RPW // _GRP


def _make_emb():
    mesh = plsc.VectorSubcoreMesh(core_axis_name="c", subcore_axis_name="s")

    @functools.partial(
        pl.kernel,
        mesh=mesh,
        out_type=jax.ShapeDtypeStruct((BATCH, SEQ, EMBED), jnp.float32),
        compiler_params=pltpu.CompilerParams(use_tc_tiling_on_sc=False),
        scratch_types=[
            pltpu.VMEM((_BPW,), jnp.int32),
            pltpu.VMEM((2 * _GRP, SEQ, EMBED), jnp.float32),
            pltpu.SemaphoreType.DMA,
            pltpu.SemaphoreType.DMA,
        ],
    )
    def emb(idx_hbm, table_hbm, out_hbm, idx_v, bufs, gsem, wsem):
        wid = lax.axis_index("s") * _NC + lax.axis_index("c")
        base = wid * _BPW
        row0 = wid * _RPW
        pltpu.sync_copy(idx_hbm.at[pl.ds(base, _BPW)], idx_v)

        def gstart(r, b):
            pltpu.make_async_copy(
                table_hbm.at[idx_v.at[pl.ds(r * SEQ, SEQ)]], bufs.at[b],
                gsem).start()

        def gwait(b):
            pltpu.make_async_copy(
                table_hbm.at[idx_v.at[pl.ds(0, SEQ)]], bufs.at[b],
                gsem).wait()

        def wstart(r, b):
            pltpu.make_async_copy(
                bufs.at[b], out_hbm.at[row0 + r], wsem).start()

        def wwait(b):
            pltpu.make_async_copy(
                bufs.at[b], out_hbm.at[row0], wsem).wait()

        def g_start(g, bb):
            for b in range(_GRP):
                gstart(g * _GRP + b, bb + b)

        def g_wait(bb):
            for b in range(_GRP):
                gwait(bb + b)

        def w_start(g, bb):
            for b in range(_GRP):
                wstart(g * _GRP + b, bb + b)

        def w_wait(bb):
            for b in range(_GRP):
                wwait(bb + b)

        g_start(0, 0)
        g_wait(0)
        w_start(0, 0)
        g_start(1, _GRP)

        def body(j, carry):
            g1 = 2 * j + 1
            g_wait(_GRP)
            w_wait(0)
            w_start(g1, _GRP)
            g_start(g1 + 1, 0)
            g2 = g1 + 1
            g_wait(0)
            w_wait(_GRP)
            w_start(g2, 0)
            g_start(g2 + 1, _GRP)
            return carry

        lax.fori_loop(0, (_NG - 2) // 2, body, 0)
        g_wait(_GRP)
        w_wait(0)
        w_start(_NG - 1, _GRP)
        w_wait(_GRP)

    return emb


_emb = _make_emb()


def kernel(input, word_embed):
    idx = input.reshape(-1).astype(jnp.int32)
    tab = with_layout_constraint(word_embed, Layout((0, 1), tiling=((8,),)))
    return _emb(idx, tab)
